# baseline (speedup 1.0000x reference)
import functools

import jax
import jax.numpy as jnp
from jax import lax
from jax.experimental import pallas as pl
from jax.experimental.pallas import tpu as pltpu
from jax.experimental.pallas import tpu_sc as plsc

NUM_CLASSES = 64
L, B, D = 4, 4096, 2048
N = L * B
N_SC = N // 2
N_TC = N - N_SC
NC, NS, LANES = 2, 16, 16
NSTRIPE = 16
DCOL = D // NSTRIPE
NROWG = 2
ROWS_W = N_SC // NROWG
R = 128
NCHUNK = ROWS_W // R
DVEC = DCOL // LANES
ACC = NUM_CLASSES * DCOL
TCB = 512
TCOFF = N_SC // TCB


def _make_sc_kernel():
    mesh = plsc.VectorSubcoreMesh(core_axis_name="c", subcore_axis_name="s")

    @functools.partial(
        pl.kernel,
        mesh=mesh,
        compiler_params=pltpu.CompilerParams(
            needs_layout_passes=False, use_tc_tiling_on_sc=False),
        out_type=[
            jax.ShapeDtypeStruct((NROWG, NUM_CLASSES, D), jnp.float32),
            jax.ShapeDtypeStruct((NROWG, NUM_CLASSES, D), jnp.float32),
            jax.ShapeDtypeStruct((NROWG, NUM_CLASSES, LANES), jnp.float32),
        ],
        scratch_types=[
            pltpu.VMEM((ACC,), jnp.float32),
            pltpu.VMEM((ACC,), jnp.float32),
            pltpu.VMEM((NUM_CLASSES * LANES,), jnp.float32),
            pltpu.VMEM((2, R, DCOL), jnp.float32),
            pltpu.VMEM((2, R, DCOL), jnp.float32),
            pltpu.VMEM((2, R), jnp.int32),
            pltpu.SemaphoreType.DMA((2,)),
            pltpu.VMEM((NUM_CLASSES, DCOL), jnp.float32),
            pltpu.VMEM((NUM_CLASSES, LANES), jnp.float32),
        ],
    )
    def seg_sum(fp_hbm, bp_hbm, lab_hbm, out_fp, out_bp, out_cnt,
                acc_fp, acc_bp, acc_cnt, rowbuf_fp, rowbuf_bp, idxbuf,
                sems, resbuf, cntres):
        cid = lax.axis_index("c")
        sid = lax.axis_index("s")
        stripe = cid * (NSTRIPE // NC) + sid // NROWG
        half = sid % NROWG
        col0 = stripe * DCOL
        row0 = half * ROWS_W

        ones_v = jnp.ones((LANES,), jnp.float32)
        zero_v = jnp.zeros((LANES,), jnp.float32)
        iota_v = lax.iota(jnp.int32, LANES)

        def zero_body(i, _):
            acc_fp[pl.ds(i * LANES, LANES)] = zero_v
            acc_bp[pl.ds(i * LANES, LANES)] = zero_v
            return 0
        lax.fori_loop(0, ACC // LANES, zero_body, 0)

        def zero_cnt_body(i, _):
            acc_cnt[pl.ds(i * LANES, LANES)] = zero_v
            return 0
        lax.fori_loop(0, NUM_CLASSES, zero_cnt_body, 0)

        def issue(b, k):
            r0 = row0 + k * R
            pltpu.async_copy(lab_hbm.at[pl.ds(r0, R)], idxbuf.at[b],
                             sems.at[b])
            pltpu.async_copy(fp_hbm.at[pl.ds(r0, R), pl.ds(col0, DCOL)],
                             rowbuf_fp.at[b], sems.at[b])
            pltpu.async_copy(bp_hbm.at[pl.ds(r0, R), pl.ds(col0, DCOL)],
                             rowbuf_bp.at[b], sems.at[b])

        def wait(b):
            pltpu.make_async_copy(lab_hbm.at[pl.ds(0, R)], idxbuf.at[b],
                                  sems.at[b]).wait()
            pltpu.make_async_copy(fp_hbm.at[pl.ds(0, R), pl.ds(0, DCOL)],
                                  rowbuf_fp.at[b], sems.at[b]).wait()
            pltpu.make_async_copy(bp_hbm.at[pl.ds(0, R), pl.ds(0, DCOL)],
                                  rowbuf_bp.at[b], sems.at[b]).wait()

        def compute(b):
            @plsc.parallel_loop(0, R // LANES, unroll=2)
            def group_body(g):
                lv = idxbuf[b, pl.ds(g * LANES, LANES)]
                plsc.addupdate_scatter(acc_cnt, [lv * LANES + iota_v],
                                       ones_v)

                def row_loads(kk):
                    i = g * LANES + kk
                    return (
                        [rowbuf_fp[b, i, pl.ds(j * LANES, LANES)]
                         for j in range(DVEC)],
                        [rowbuf_bp[b, i, pl.ds(j * LANES, LANES)]
                         for j in range(DVEC)],
                    )

                fvals, bvals = row_loads(0)
                for kk in range(LANES):
                    off = lv[kk] * DCOL
                    nxt = row_loads(kk + 1) if kk + 1 < LANES else None
                    for j in range(DVEC):
                        plsc.addupdate(
                            acc_fp.at[pl.ds(off + (j * LANES), LANES)],
                            fvals[j])
                    for j in range(DVEC):
                        plsc.addupdate(
                            acc_bp.at[pl.ds(off + (j * LANES), LANES)],
                            bvals[j])
                    if nxt is not None:
                        fvals, bvals = nxt

        issue(0, 0)
        issue(1, 1)

        def chunk_body(t, _):
            for b in range(2):
                k = 2 * t + b
                wait(b)
                compute(b)
                issue(b, k + 2)
            return 0
        lax.fori_loop(0, NCHUNK // 2 - 1, chunk_body, 0)
        for b in range(2):
            wait(b)
            compute(b)

        def emit_body(r, _):
            for j in range(DVEC):
                resbuf[r, pl.ds(j * LANES, LANES)] = (
                    acc_fp[pl.ds(r * DCOL + j * LANES, LANES)])
            return 0
        lax.fori_loop(0, NUM_CLASSES, emit_body, 0)
        pltpu.sync_copy(resbuf, out_fp.at[half, :, pl.ds(col0, DCOL)])

        def emit_body_bp(r, _):
            for j in range(DVEC):
                resbuf[r, pl.ds(j * LANES, LANES)] = (
                    acc_bp[pl.ds(r * DCOL + j * LANES, LANES)])
            return 0
        lax.fori_loop(0, NUM_CLASSES, emit_body_bp, 0)
        pltpu.sync_copy(resbuf, out_bp.at[half, :, pl.ds(col0, DCOL)])

        def emit_cnt(r, _):
            cntres[r, :] = acc_cnt[pl.ds(r * LANES, LANES)]
            return 0
        lax.fori_loop(0, NUM_CLASSES, emit_cnt, 0)
        pltpu.sync_copy(cntres, out_cnt.at[half])

    return seg_sum


def _tc_body(lab_ref, fp_ref, bp_ref, ofp, obp, ocnt):
    i = pl.program_id(0)
    lab = lab_ref[0, 0, :]
    oh = (lax.broadcasted_iota(jnp.int32, (NUM_CLASSES, TCB), 0)
          == lab[None, :]).astype(jnp.float32)
    pfp = jnp.dot(oh, fp_ref[...], preferred_element_type=jnp.float32)
    pbp = jnp.dot(oh, bp_ref[...], preferred_element_type=jnp.float32)
    pc = jnp.broadcast_to(jnp.sum(oh, axis=1, keepdims=True),
                          (NUM_CLASSES, 128))

    @pl.when(i == 0)
    def _():
        ofp[...] = pfp
        obp[...] = pbp
        ocnt[...] = pc

    @pl.when(i > 0)
    def _():
        ofp[...] = ofp[...] + pfp
        obp[...] = obp[...] + pbp
        ocnt[...] = ocnt[...] + pc


_TC_PARTIAL = pl.pallas_call(
    _tc_body,
    grid=(N_TC // TCB,),
    in_specs=[
        pl.BlockSpec((1, 1, TCB), lambda i: (TCOFF + i, 0, 0)),
        pl.BlockSpec((TCB, D), lambda i: (TCOFF + i, 0)),
        pl.BlockSpec((TCB, D), lambda i: (TCOFF + i, 0)),
    ],
    out_specs=[
        pl.BlockSpec((NUM_CLASSES, D), lambda i: (0, 0)),
        pl.BlockSpec((NUM_CLASSES, D), lambda i: (0, 0)),
        pl.BlockSpec((NUM_CLASSES, 128), lambda i: (0, 0)),
    ],
    out_shape=[
        jax.ShapeDtypeStruct((NUM_CLASSES, D), jnp.float32),
        jax.ShapeDtypeStruct((NUM_CLASSES, D), jnp.float32),
        jax.ShapeDtypeStruct((NUM_CLASSES, 128), jnp.float32),
    ],
)


def _comb_body(scf_ref, scb_ref, scc_ref, tf_ref, tb_ref, tcnt_ref,
               ofp, obp):
    c = (jnp.sum(scc_ref[0], axis=1) + jnp.sum(scc_ref[1], axis=1)
         + tcnt_ref[:, 0])
    rec = (1.0 / jnp.maximum(c, 1.0))[:, None]
    ofp[...] = (scf_ref[0] + scf_ref[1] + tf_ref[...]) * rec
    obp[...] = (scb_ref[0] + scb_ref[1] + tb_ref[...]) * rec


_COMBINE = pl.pallas_call(
    _comb_body,
    out_shape=[
        jax.ShapeDtypeStruct((NUM_CLASSES, D), jnp.float32),
        jax.ShapeDtypeStruct((NUM_CLASSES, D), jnp.float32),
    ],
)

_SEG_SUM_SC = _make_sc_kernel()


def kernel(s_fp_list, s_bp_list, classes):
    fp = s_fp_list.reshape(N, D)
    bp = s_bp_list.reshape(N, D)
    labels = classes.reshape(B, L).T.reshape(N)
    sc_fp, sc_bp, sc_cnt = _SEG_SUM_SC(fp, bp, labels)
    tc_fp, tc_bp, tc_cnt = _TC_PARTIAL(
        labels.reshape(N // TCB, 1, TCB), fp, bp)
    out_fp, out_bp = _COMBINE(sc_fp, sc_bp, sc_cnt, tc_fp, tc_bp, tc_cnt)
    return (out_fp, out_bp)


# speedup vs baseline: 1.3778x; 1.3778x over previous
import functools

import jax
import jax.numpy as jnp
from jax import lax
from jax.experimental import pallas as pl
from jax.experimental.pallas import tpu as pltpu
from jax.experimental.pallas import tpu_sc as plsc

NUM_CLASSES = 64
L, B, D = 4, 4096, 2048
N = L * B
NC, NS, LANES = 2, 16, 16
NSTRIPE = 16
DCOL = D // NSTRIPE
NROWG = 2
ROWS_W = N // NROWG
R = 128
NCHUNK = ROWS_W // R
DVEC = DCOL // LANES
ACC = NUM_CLASSES * DCOL
CCLS = NUM_CLASSES // NROWG
HALFACC = CCLS * DCOL


def _make_sc_kernel():
    mesh = plsc.VectorSubcoreMesh(core_axis_name="c", subcore_axis_name="s")

    @functools.partial(
        pl.kernel,
        mesh=mesh,
        compiler_params=pltpu.CompilerParams(
            needs_layout_passes=False, use_tc_tiling_on_sc=False),
        out_type=[
            jax.ShapeDtypeStruct((NUM_CLASSES, D), jnp.float32),
            jax.ShapeDtypeStruct((NUM_CLASSES, D), jnp.float32),
        ],
        scratch_types=[
            pltpu.VMEM_SHARED((NS, ACC), jnp.float32),
            pltpu.VMEM_SHARED((NS, ACC), jnp.float32),
            pltpu.VMEM_SHARED((NS, NUM_CLASSES * LANES), jnp.float32),
            pltpu.VMEM((ACC,), jnp.float32),
            pltpu.VMEM((ACC,), jnp.float32),
            pltpu.VMEM((NUM_CLASSES * LANES,), jnp.float32),
            pltpu.VMEM((2, R, DCOL), jnp.float32),
            pltpu.VMEM((2, R, DCOL), jnp.float32),
            pltpu.VMEM((2, R), jnp.int32),
            pltpu.SemaphoreType.DMA((2,)),
            pltpu.VMEM((HALFACC,), jnp.float32),
            pltpu.VMEM((CCLS * LANES,), jnp.float32),
            pltpu.VMEM((CCLS, DCOL), jnp.float32),
        ],
    )
    def seg_mean(fp_hbm, bp_hbm, lab_hbm, out_fp, out_bp,
                 sp_fp, sp_bp, sp_cnt, acc_fp, acc_bp, acc_cnt,
                 rowbuf_fp, rowbuf_bp, idxbuf, sems,
                 partbuf, cntpart, resbuf):
        cid = lax.axis_index("c")
        sid = lax.axis_index("s")
        stripe = cid * (NSTRIPE // NC) + sid // NROWG
        half = sid % NROWG
        col0 = stripe * DCOL
        row0 = half * ROWS_W
        partner = sid - half + (1 - half)

        ones_v = jnp.ones((LANES,), jnp.float32)
        zero_v = jnp.zeros((LANES,), jnp.float32)
        iota_v = lax.iota(jnp.int32, LANES)

        def zero_body(i, _):
            acc_fp[pl.ds(i * LANES, LANES)] = zero_v
            acc_bp[pl.ds(i * LANES, LANES)] = zero_v
            return 0
        lax.fori_loop(0, ACC // LANES, zero_body, 0)

        def zero_cnt_body(i, _):
            acc_cnt[pl.ds(i * LANES, LANES)] = zero_v
            return 0
        lax.fori_loop(0, NUM_CLASSES, zero_cnt_body, 0)

        def issue(b, k):
            r0 = row0 + k * R
            pltpu.async_copy(lab_hbm.at[pl.ds(r0, R)], idxbuf.at[b],
                             sems.at[b])
            pltpu.async_copy(fp_hbm.at[pl.ds(r0, R), pl.ds(col0, DCOL)],
                             rowbuf_fp.at[b], sems.at[b])
            pltpu.async_copy(bp_hbm.at[pl.ds(r0, R), pl.ds(col0, DCOL)],
                             rowbuf_bp.at[b], sems.at[b])

        def wait(b):
            pltpu.make_async_copy(lab_hbm.at[pl.ds(0, R)], idxbuf.at[b],
                                  sems.at[b]).wait()
            pltpu.make_async_copy(fp_hbm.at[pl.ds(0, R), pl.ds(0, DCOL)],
                                  rowbuf_fp.at[b], sems.at[b]).wait()
            pltpu.make_async_copy(bp_hbm.at[pl.ds(0, R), pl.ds(0, DCOL)],
                                  rowbuf_bp.at[b], sems.at[b]).wait()

        def compute(b):
            @plsc.parallel_loop(0, R // LANES, unroll=2)
            def group_body(g):
                lv = idxbuf[b, pl.ds(g * LANES, LANES)]
                plsc.addupdate_scatter(acc_cnt, [lv * LANES + iota_v],
                                       ones_v)
                def row_loads(kk):
                    i = g * LANES + kk
                    return (
                        [rowbuf_fp[b, i, pl.ds(j * LANES, LANES)]
                         for j in range(DVEC)],
                        [rowbuf_bp[b, i, pl.ds(j * LANES, LANES)]
                         for j in range(DVEC)],
                    )

                fvals, bvals = row_loads(0)
                for kk in range(LANES):
                    off = lv[kk] * DCOL
                    nxt = row_loads(kk + 1) if kk + 1 < LANES else None
                    for j in range(DVEC):
                        plsc.addupdate(
                            acc_fp.at[pl.ds(off + (j * LANES), LANES)],
                            fvals[j])
                    for j in range(DVEC):
                        plsc.addupdate(
                            acc_bp.at[pl.ds(off + (j * LANES), LANES)],
                            bvals[j])
                    if nxt is not None:
                        fvals, bvals = nxt

        issue(0, 0)
        issue(1, 1)

        def chunk_body(t, _):
            for b in range(2):
                k = 2 * t + b
                wait(b)
                compute(b)
                issue(b, k + 2)
            return 0
        lax.fori_loop(0, NCHUNK // 2 - 1, chunk_body, 0)
        for b in range(2):
            wait(b)
            compute(b)

        pltpu.sync_copy(acc_fp, sp_fp.at[sid])
        pltpu.sync_copy(acc_bp, sp_bp.at[sid])
        pltpu.sync_copy(acc_cnt, sp_cnt.at[sid])
        plsc.subcore_barrier()

        c0 = half * HALFACC
        n0 = half * (CCLS * LANES)
        pltpu.sync_copy(sp_cnt.at[partner, pl.ds(n0, CCLS * LANES)], cntpart)

        def finalize(acc, sp, out_hbm):
            pltpu.sync_copy(sp.at[partner, pl.ds(c0, HALFACC)], partbuf)
            for r in range(CCLS):
                cnt = (acc_cnt[pl.ds(n0 + r * LANES, LANES)]
                       + cntpart[pl.ds(r * LANES, LANES)])
                tot = jnp.broadcast_to(jnp.sum(cnt), (LANES,))
                rec = 1.0 / jnp.maximum(tot, 1.0)
                for j in range(DVEC):
                    o = r * DCOL + j * LANES
                    resbuf[r, pl.ds(j * LANES, LANES)] = (
                        (acc[pl.ds(c0 + o, LANES)]
                         + partbuf[pl.ds(o, LANES)]) * rec)
            pltpu.sync_copy(
                resbuf,
                out_hbm.at[pl.ds(half * CCLS, CCLS), pl.ds(col0, DCOL)])

        finalize(acc_fp, sp_fp, out_fp)
        finalize(acc_bp, sp_bp, out_bp)

    return seg_mean


_SEG_MEAN = _make_sc_kernel()


def kernel(s_fp_list, s_bp_list, classes):
    fp = s_fp_list.reshape(N, D)
    bp = s_bp_list.reshape(N, D)
    labels = classes.reshape(B, L).T.reshape(N)
    out_fp, out_bp = _SEG_MEAN(fp, bp, labels)
    return (out_fp, out_bp)
